# baseline (speedup 1.0000x reference)
import functools

import jax
import jax.numpy as jnp
from jax import lax
from jax.experimental import pallas as pl
from jax.experimental.pallas import tpu as pltpu
from jax.experimental.pallas import tpu_sc as plsc

N = 10000
E = 320000
H = 128
NC, NS = 2, 16
NW = NC * NS
EC = 128
CPW = 80
IH = 40
EP = NW * CPW * EC
NP = 10240
RPT = NP // NS
RC = 128
NRC = RPT // RC
CW = 16
NB = 10
BR = N // NB

def _fill_rows(rows_v, value):
    def _row(i, _):
        def _inner(j, _):
            rows_v[i, pl.ds(j * 16, 16)] = jnp.full((16,), value, jnp.float32)
            return 0
        lax.fori_loop(0, H // 16, _inner, 0)
        return 0
    lax.fori_loop(0, EC, _row, 0)


def _zero_acc_slice(rows_v, acc_sh, row0):
    _fill_rows(rows_v, 0.0)

    def _zero(k, _):
        pltpu.sync_copy(rows_v, acc_sh.at[pl.ds(row0 + k * RC, RC)])
        return 0
    lax.fori_loop(0, NRC, _zero, 0)


def _copy_out_slice(rows_v, acc_sh, out_hbm, cid, row0):
    def _out(k, _):
        r = row0 + k * RC
        pltpu.sync_copy(acc_sh.at[pl.ds(r, RC)], rows_v)
        pltpu.sync_copy(rows_v, out_hbm.at[cid, pl.ds(r, RC)])
        return 0
    lax.fori_loop(0, NRC, _out, 0)


def _sc_body(z_hbm, src_hbm, dst_hbm, agg_out,
             src_v, dst_v, rows_v, acc_sh):
    cid = lax.axis_index("c")
    sid = lax.axis_index("s")
    wid = sid * NC + cid
    row0 = sid * RPT

    _zero_acc_slice(rows_v, acc_sh, row0)
    plsc.subcore_barrier()

    for h in range(CPW // IH):
        pltpu.sync_copy(src_hbm.at[pl.ds(wid * CPW + h * IH, IH)], src_v)
        pltpu.sync_copy(dst_hbm.at[pl.ds(wid * CPW + h * IH, IH)], dst_v)

        @pl.loop(0, IH)
        def _chunk(k):
            pltpu.sync_copy(z_hbm.at[src_v.at[k]], rows_v)
            pltpu.sync_copy(rows_v, acc_sh.at[dst_v.at[k]], add=True)

    plsc.subcore_barrier()
    _copy_out_slice(rows_v, acc_sh, agg_out, cid, row0)


def _sc_count_body(dst_hbm, cnt_out, dst_v, rows_v, acc_sh):
    cid = lax.axis_index("c")
    sid = lax.axis_index("s")
    wid = sid * NC + cid
    row0 = sid * RPT

    _zero_acc_slice(rows_v, acc_sh, row0)
    _fill_rows(rows_v, 1.0)
    plsc.subcore_barrier()

    for h in range(CPW // IH):
        pltpu.sync_copy(dst_hbm.at[pl.ds(wid * CPW + h * IH, IH)], dst_v)

        @pl.loop(0, IH)
        def _chunk(k):
            pltpu.sync_copy(rows_v, acc_sh.at[dst_v.at[k]], add=True)

    plsc.subcore_barrier()
    _copy_out_slice(rows_v, acc_sh, cnt_out, cid, row0)


def _sc_mesh():
    return plsc.VectorSubcoreMesh(
        core_axis_name="c", subcore_axis_name="s",
        num_cores=NC, num_subcores=NS,
    )


@functools.cache
def _sc_kernel():
    return pl.kernel(
        _sc_body,
        out_type=jax.ShapeDtypeStruct((NC, NP, H), jnp.float32),
        mesh=_sc_mesh(),
        scratch_types=[
            pltpu.VMEM((IH, EC), jnp.int32),
            pltpu.VMEM((IH, EC), jnp.int32),
            pltpu.VMEM((EC, H), jnp.float32),
            pltpu.VMEM_SHARED((NP, H), jnp.float32),
        ],
    )


@functools.cache
def _sc_count_kernel():
    return pl.kernel(
        _sc_count_body,
        out_type=jax.ShapeDtypeStruct((NC, NP, H), jnp.float32),
        mesh=_sc_mesh(),
        scratch_types=[
            pltpu.VMEM((IH, EC), jnp.int32),
            pltpu.VMEM((EC, H), jnp.float32),
            pltpu.VMEM_SHARED((NP, H), jnp.float32),
        ],
    )


def _sc_aggregate(z, src, dst):
    return _sc_kernel()(z, src, dst)


def _sc_count(dst):
    return _sc_count_kernel()(dst)


def _enc_body(x_ref, w1_ref, b1_ref, w2_ref, b2_ref, o_ref):
    h = jnp.dot(x_ref[...], w1_ref[...], preferred_element_type=jnp.float32)
    h = jnp.maximum(h + b1_ref[...], 0.0)
    o_ref[...] = (
        jnp.dot(h, w2_ref[...], preferred_element_type=jnp.float32)
        + b2_ref[...]
    )


def _encode(xin, w1, b1, w2, b2):
    return pl.pallas_call(
        _enc_body,
        grid=(NB,),
        in_specs=[
            pl.BlockSpec((BR, xin.shape[1]), lambda i: (i, 0)),
            pl.BlockSpec(w1.shape, lambda i: (0, 0)),
            pl.BlockSpec(b1.shape, lambda i: (0, 0)),
            pl.BlockSpec(w2.shape, lambda i: (0, 0)),
            pl.BlockSpec(b2.shape, lambda i: (0, 0)),
        ],
        out_specs=pl.BlockSpec((BR, H), lambda i: (i, 0)),
        out_shape=jax.ShapeDtypeStruct((N, H), jnp.float32),
    )(xin, w1, b1, w2, b2)


def _mean_agg(p_ref, c_ref):
    cnt = c_ref[0][:, 0:1] + c_ref[1][:, 0:1]
    recip = 1.0 / jnp.maximum(cnt, 1.0)
    return (p_ref[0] + p_ref[1]) * recip


def _layer_body(p_ref, c_ref, z_ref, wl_ref, bl_ref, wr_ref, g_ref, b_ref,
                o_ref, t_buf, s_ref, q_ref):
    i = pl.program_id(0)

    @pl.when(i == 0)
    def _():
        s_ref[...] = jnp.zeros_like(s_ref)
        q_ref[...] = jnp.zeros_like(q_ref)

    @pl.when(i < NB)
    def _():
        agg = _mean_agg(p_ref, c_ref)
        t = (
            jnp.dot(agg, wl_ref[...], preferred_element_type=jnp.float32)
            + bl_ref[...]
            + jnp.dot(z_ref[...], wr_ref[...], preferred_element_type=jnp.float32)
        )
        t_buf[pl.ds(i * BR, BR), :] = t
        s_ref[0:1, :] += jnp.sum(t, axis=0, keepdims=True)
        q_ref[0:1, :] += jnp.sum(t * t, axis=0, keepdims=True)

    @pl.when(i >= NB)
    def _():
        j = i - NB
        t = t_buf[pl.ds(j * BR, BR), :]
        m = s_ref[0:1, :] * (1.0 / N)
        v = q_ref[0:1, :] * (1.0 / N) - m * m
        o_ref[...] = jnp.maximum(
            (t - m) * lax.rsqrt(v + 1e-5) * g_ref[...] + b_ref[...], 0.0
        )


def _layer(p, c, z, wl, bl, wr, g, b):
    return pl.pallas_call(
        _layer_body,
        grid=(2 * NB,),
        in_specs=[
            pl.BlockSpec((NC, BR, H), lambda i: (0, jnp.minimum(i, NB - 1), 0)),
            pl.BlockSpec((NC, BR, H), lambda i: (0, jnp.minimum(i, NB - 1), 0)),
            pl.BlockSpec((BR, H), lambda i: (jnp.minimum(i, NB - 1), 0)),
            pl.BlockSpec(wl.shape, lambda i: (0, 0)),
            pl.BlockSpec(bl.shape, lambda i: (0, 0)),
            pl.BlockSpec(wr.shape, lambda i: (0, 0)),
            pl.BlockSpec(g.shape, lambda i: (0, 0)),
            pl.BlockSpec(b.shape, lambda i: (0, 0)),
        ],
        out_specs=pl.BlockSpec((BR, H), lambda i: (jnp.maximum(i - NB, 0), 0)),
        out_shape=jax.ShapeDtypeStruct((N, H), jnp.float32),
        scratch_shapes=[
            pltpu.VMEM((N, H), jnp.float32),
            pltpu.VMEM((8, H), jnp.float32),
            pltpu.VMEM((8, H), jnp.float32),
        ],
    )(p, c, z, wl, bl, wr, g, b)


def _final_body(p_ref, c_ref, z_ref, wl_ref, bl_ref, wr_ref,
                w1_ref, b1_ref, w2_ref, b2_ref, o_ref):
    agg = _mean_agg(p_ref, c_ref)
    t = (
        jnp.dot(agg, wl_ref[...], preferred_element_type=jnp.float32)
        + bl_ref[...]
        + jnp.dot(z_ref[...], wr_ref[...], preferred_element_type=jnp.float32)
    )
    h = jnp.dot(t, w1_ref[...], preferred_element_type=jnp.float32)
    h = jnp.maximum(h + b1_ref[...], 0.0)
    o_ref[...] = (
        jnp.dot(h, w2_ref[...], preferred_element_type=jnp.float32)
        + b2_ref[...]
    )


def _final(p, c, z, wl, bl, wr, w1, b1, w2, b2):
    od = w2.shape[1]
    return pl.pallas_call(
        _final_body,
        grid=(NB,),
        in_specs=[
            pl.BlockSpec((NC, BR, H), lambda i: (0, i, 0)),
            pl.BlockSpec((NC, BR, H), lambda i: (0, i, 0)),
            pl.BlockSpec((BR, H), lambda i: (i, 0)),
            pl.BlockSpec(wl.shape, lambda i: (0, 0)),
            pl.BlockSpec(bl.shape, lambda i: (0, 0)),
            pl.BlockSpec(wr.shape, lambda i: (0, 0)),
            pl.BlockSpec(w1.shape, lambda i: (0, 0)),
            pl.BlockSpec(b1.shape, lambda i: (0, 0)),
            pl.BlockSpec(w2.shape, lambda i: (0, 0)),
            pl.BlockSpec(b2.shape, lambda i: (0, 0)),
        ],
        out_specs=pl.BlockSpec((BR, od), lambda i: (i, 0)),
        out_shape=jax.ShapeDtypeStruct((N, od), jnp.float32),
    )(p, c, z, wl, bl, wr, w1, b1, w2, b2)


def kernel(x, fx, enc_W1, enc_b1, enc_W2, enc_b2, sage_Wl, sage_bl, sage_Wr,
           bn_g, bn_b, dec_W1, dec_b1, dec_W2, dec_b2, geo):
    xin = jnp.concatenate([x[0], fx[0]], axis=-1)
    pad = EP - E
    src = jnp.concatenate(
        [geo[0], jnp.zeros((pad,), jnp.int32)]).reshape(EP // EC, EC)
    dst = jnp.concatenate(
        [geo[1], N + (jnp.arange(pad, dtype=jnp.int32) % (NP - N))]
    ).reshape(EP // EC, EC)
    z = _encode(xin, enc_W1, enc_b1.reshape(1, -1), enc_W2,
                enc_b2.reshape(1, -1))
    c = _sc_count(dst)
    n_layers = sage_Wl.shape[0] - 1
    for l in range(n_layers):
        p = _sc_aggregate(z, src, dst)
        z = _layer(p, c, z, sage_Wl[l], sage_bl[l].reshape(1, -1),
                   sage_Wr[l], bn_g[l].reshape(1, -1),
                   bn_b[l].reshape(1, -1))
    p = _sc_aggregate(z, src, dst)
    out = _final(p, c, z, sage_Wl[n_layers], sage_bl[n_layers].reshape(1, -1),
                 sage_Wr[n_layers], dec_W1, dec_b1.reshape(1, -1),
                 dec_W2, dec_b2.reshape(1, -1))
    return out[None]


# speedup vs baseline: 1.8923x; 1.8923x over previous
import functools

import jax
import jax.numpy as jnp
from jax import lax
from jax.experimental import pallas as pl
from jax.experimental.pallas import tpu as pltpu
from jax.experimental.pallas import tpu_sc as plsc

N = 10000
E = 320000
H = 128
NC, NS = 2, 16
NW = NC * NS
EC = 128
NCHUNK = E // EC
FULL_K = NCHUNK // NW
TAIL = NCHUNK - FULL_K * NW
NP = 10240
RPT = NP // NS
RC = 128
NRC = RPT // RC
CW = 16
NB = 10
BR = N // NB

def _fill_rows(rows_v, value):
    def _row(i, _):
        def _inner(j, _):
            rows_v[i, pl.ds(j * 16, 16)] = jnp.full((16,), value, jnp.float32)
            return 0
        lax.fori_loop(0, H // 16, _inner, 0)
        return 0
    lax.fori_loop(0, EC, _row, 0)


def _zero_acc_slice(rows_v, acc_sh, row0):
    _fill_rows(rows_v, 0.0)

    def _zero(k, _):
        pltpu.sync_copy(rows_v, acc_sh.at[pl.ds(row0 + k * RC, RC)])
        return 0
    lax.fori_loop(0, NRC, _zero, 0)


def _copy_out_slice(rows_v, acc_sh, out_hbm, cid, row0):
    def _out(k, _):
        r = row0 + k * RC
        pltpu.sync_copy(acc_sh.at[pl.ds(r, RC)], rows_v)
        pltpu.sync_copy(rows_v, out_hbm.at[cid, pl.ds(r, RC)])
        return 0
    lax.fori_loop(0, NRC, _out, 0)


def _sc_body(z_hbm, src_hbm, dst_hbm, agg_out,
             src_v, dst_v, rows_v, acc_sh, sem):
    cid = lax.axis_index("c")
    sid = lax.axis_index("s")
    wid = sid * NC + cid
    row0 = sid * RPT

    _zero_acc_slice(rows_v, acc_sh, row0)
    plsc.subcore_barrier()

    def _chunk_body(q):
        base = q * EC
        pltpu.sync_copy(src_hbm.at[pl.ds(base, EC)], src_v.at[0])
        pltpu.sync_copy(dst_hbm.at[pl.ds(base, EC)], dst_v.at[0])
        pltpu.async_copy(z_hbm.at[src_v.at[0]], rows_v, sem).wait()
        pltpu.sync_copy(rows_v, acc_sh.at[dst_v.at[0]], add=True)

    def _chunk(k, _):
        _chunk_body(wid * FULL_K + k)
        return 0
    lax.fori_loop(0, FULL_K, _chunk, 0)

    @pl.when(wid < TAIL)
    def _():
        _chunk_body(NW * FULL_K + wid)

    plsc.subcore_barrier()
    _copy_out_slice(rows_v, acc_sh, agg_out, cid, row0)


def _sc_count_body(dst_hbm, cnt_out, dst_v, rows_v, acc_sh):
    cid = lax.axis_index("c")
    sid = lax.axis_index("s")
    wid = sid * NC + cid
    row0 = sid * RPT

    _zero_acc_slice(rows_v, acc_sh, row0)
    _fill_rows(rows_v, 1.0)
    plsc.subcore_barrier()

    def _chunk_body(q):
        pltpu.sync_copy(dst_hbm.at[pl.ds(q * EC, EC)], dst_v.at[0])
        pltpu.sync_copy(rows_v, acc_sh.at[dst_v.at[0]], add=True)

    def _chunk(k, _):
        _chunk_body(wid * FULL_K + k)
        return 0
    lax.fori_loop(0, FULL_K, _chunk, 0)

    @pl.when(wid < TAIL)
    def _():
        _chunk_body(NW * FULL_K + wid)

    plsc.subcore_barrier()
    _copy_out_slice(rows_v, acc_sh, cnt_out, cid, row0)


def _sc_mesh():
    return plsc.VectorSubcoreMesh(
        core_axis_name="c", subcore_axis_name="s",
        num_cores=NC, num_subcores=NS,
    )


@functools.cache
def _sc_kernel():
    return pl.kernel(
        _sc_body,
        out_type=jax.ShapeDtypeStruct((NC, NP, H), jnp.float32),
        mesh=_sc_mesh(),
        scratch_types=[
            pltpu.VMEM((1, EC), jnp.int32),
            pltpu.VMEM((1, EC), jnp.int32),
            pltpu.VMEM((EC, H), jnp.float32),
            pltpu.VMEM_SHARED((NP, H), jnp.float32),
            pltpu.SemaphoreType.DMA,
        ],
    )


@functools.cache
def _sc_count_kernel():
    return pl.kernel(
        _sc_count_body,
        out_type=jax.ShapeDtypeStruct((NC, NP, H), jnp.float32),
        mesh=_sc_mesh(),
        scratch_types=[
            pltpu.VMEM((1, EC), jnp.int32),
            pltpu.VMEM((EC, H), jnp.float32),
            pltpu.VMEM_SHARED((NP, H), jnp.float32),
        ],
    )


def _sc_aggregate(z, src, dst):
    return _sc_kernel()(z, src, dst)


def _sc_count(dst):
    return _sc_count_kernel()(dst)


def _enc_body(x_ref, w1_ref, b1_ref, w2_ref, b2_ref, o_ref):
    h = jnp.dot(x_ref[...], w1_ref[...], preferred_element_type=jnp.float32)
    h = jnp.maximum(h + b1_ref[...], 0.0)
    o_ref[...] = (
        jnp.dot(h, w2_ref[...], preferred_element_type=jnp.float32)
        + b2_ref[...]
    )


def _encode(xin, w1, b1, w2, b2):
    return pl.pallas_call(
        _enc_body,
        grid=(NB,),
        in_specs=[
            pl.BlockSpec((BR, xin.shape[1]), lambda i: (i, 0)),
            pl.BlockSpec(w1.shape, lambda i: (0, 0)),
            pl.BlockSpec(b1.shape, lambda i: (0, 0)),
            pl.BlockSpec(w2.shape, lambda i: (0, 0)),
            pl.BlockSpec(b2.shape, lambda i: (0, 0)),
        ],
        out_specs=pl.BlockSpec((BR, H), lambda i: (i, 0)),
        out_shape=jax.ShapeDtypeStruct((N, H), jnp.float32),
    )(xin, w1, b1, w2, b2)


def _mean_agg(p_ref, c_ref):
    cnt = c_ref[0][:, 0:1] + c_ref[1][:, 0:1]
    recip = 1.0 / jnp.maximum(cnt, 1.0)
    return (p_ref[0] + p_ref[1]) * recip


def _layer_body(p_ref, c_ref, z_ref, wl_ref, bl_ref, wr_ref, g_ref, b_ref,
                o_ref, t_buf, s_ref, q_ref):
    i = pl.program_id(0)

    @pl.when(i == 0)
    def _():
        s_ref[...] = jnp.zeros_like(s_ref)
        q_ref[...] = jnp.zeros_like(q_ref)

    @pl.when(i < NB)
    def _():
        agg = _mean_agg(p_ref, c_ref)
        t = (
            jnp.dot(agg, wl_ref[...], preferred_element_type=jnp.float32)
            + bl_ref[...]
            + jnp.dot(z_ref[...], wr_ref[...], preferred_element_type=jnp.float32)
        )
        t_buf[pl.ds(i * BR, BR), :] = t
        s_ref[0:1, :] += jnp.sum(t, axis=0, keepdims=True)
        q_ref[0:1, :] += jnp.sum(t * t, axis=0, keepdims=True)

    @pl.when(i >= NB)
    def _():
        j = i - NB
        t = t_buf[pl.ds(j * BR, BR), :]
        m = s_ref[0:1, :] * (1.0 / N)
        v = q_ref[0:1, :] * (1.0 / N) - m * m
        o_ref[...] = jnp.maximum(
            (t - m) * lax.rsqrt(v + 1e-5) * g_ref[...] + b_ref[...], 0.0
        )


def _layer(p, c, z, wl, bl, wr, g, b):
    return pl.pallas_call(
        _layer_body,
        grid=(2 * NB,),
        in_specs=[
            pl.BlockSpec((NC, BR, H), lambda i: (0, jnp.minimum(i, NB - 1), 0)),
            pl.BlockSpec((NC, BR, H), lambda i: (0, jnp.minimum(i, NB - 1), 0)),
            pl.BlockSpec((BR, H), lambda i: (jnp.minimum(i, NB - 1), 0)),
            pl.BlockSpec(wl.shape, lambda i: (0, 0)),
            pl.BlockSpec(bl.shape, lambda i: (0, 0)),
            pl.BlockSpec(wr.shape, lambda i: (0, 0)),
            pl.BlockSpec(g.shape, lambda i: (0, 0)),
            pl.BlockSpec(b.shape, lambda i: (0, 0)),
        ],
        out_specs=pl.BlockSpec((BR, H), lambda i: (jnp.maximum(i - NB, 0), 0)),
        out_shape=jax.ShapeDtypeStruct((N, H), jnp.float32),
        scratch_shapes=[
            pltpu.VMEM((N, H), jnp.float32),
            pltpu.VMEM((8, H), jnp.float32),
            pltpu.VMEM((8, H), jnp.float32),
        ],
    )(p, c, z, wl, bl, wr, g, b)


def _final_body(p_ref, c_ref, z_ref, wl_ref, bl_ref, wr_ref,
                w1_ref, b1_ref, w2_ref, b2_ref, o_ref):
    agg = _mean_agg(p_ref, c_ref)
    t = (
        jnp.dot(agg, wl_ref[...], preferred_element_type=jnp.float32)
        + bl_ref[...]
        + jnp.dot(z_ref[...], wr_ref[...], preferred_element_type=jnp.float32)
    )
    h = jnp.dot(t, w1_ref[...], preferred_element_type=jnp.float32)
    h = jnp.maximum(h + b1_ref[...], 0.0)
    o_ref[...] = (
        jnp.dot(h, w2_ref[...], preferred_element_type=jnp.float32)
        + b2_ref[...]
    )


def _final(p, c, z, wl, bl, wr, w1, b1, w2, b2):
    od = w2.shape[1]
    return pl.pallas_call(
        _final_body,
        grid=(NB,),
        in_specs=[
            pl.BlockSpec((NC, BR, H), lambda i: (0, i, 0)),
            pl.BlockSpec((NC, BR, H), lambda i: (0, i, 0)),
            pl.BlockSpec((BR, H), lambda i: (i, 0)),
            pl.BlockSpec(wl.shape, lambda i: (0, 0)),
            pl.BlockSpec(bl.shape, lambda i: (0, 0)),
            pl.BlockSpec(wr.shape, lambda i: (0, 0)),
            pl.BlockSpec(w1.shape, lambda i: (0, 0)),
            pl.BlockSpec(b1.shape, lambda i: (0, 0)),
            pl.BlockSpec(w2.shape, lambda i: (0, 0)),
            pl.BlockSpec(b2.shape, lambda i: (0, 0)),
        ],
        out_specs=pl.BlockSpec((BR, od), lambda i: (i, 0)),
        out_shape=jax.ShapeDtypeStruct((N, od), jnp.float32),
    )(p, c, z, wl, bl, wr, w1, b1, w2, b2)


def kernel(x, fx, enc_W1, enc_b1, enc_W2, enc_b2, sage_Wl, sage_bl, sage_Wr,
           bn_g, bn_b, dec_W1, dec_b1, dec_W2, dec_b2, geo):
    xin = jnp.concatenate([x[0], fx[0]], axis=-1)
    src = geo[0]
    dst = geo[1]
    z = _encode(xin, enc_W1, enc_b1.reshape(1, -1), enc_W2,
                enc_b2.reshape(1, -1))
    c = _sc_count(dst)
    n_layers = sage_Wl.shape[0] - 1
    for l in range(n_layers):
        p = _sc_aggregate(z, src, dst)
        z = _layer(p, c, z, sage_Wl[l], sage_bl[l].reshape(1, -1),
                   sage_Wr[l], bn_g[l].reshape(1, -1),
                   bn_b[l].reshape(1, -1))
    p = _sc_aggregate(z, src, dst)
    out = _final(p, c, z, sage_Wl[n_layers], sage_bl[n_layers].reshape(1, -1),
                 sage_Wr[n_layers], dec_W1, dec_b1.reshape(1, -1),
                 dec_W2, dec_b2.reshape(1, -1))
    return out[None]


# speedup vs baseline: 2.8141x; 1.4871x over previous
import functools

import jax
import jax.numpy as jnp
from jax import lax
from jax.experimental import pallas as pl
from jax.experimental.pallas import tpu as pltpu
from jax.experimental.pallas import tpu_sc as plsc

N = 10000
E = 320000
H = 128
NC, NS = 2, 16
NW = NC * NS
EC = 128
NCHUNK = E // EC
FULL_K = NCHUNK // NW
TAIL = NCHUNK - FULL_K * NW
NP = 10240
RPT = NP // NS
RC = 128
NRC = RPT // RC
CW = 16
NB = 10
BR = N // NB

def _fill_rows(rows_v, value):
    def _row(i, _):
        def _inner(j, _):
            rows_v[i, pl.ds(j * 16, 16)] = jnp.full((16,), value, jnp.float32)
            return 0
        lax.fori_loop(0, H // 16, _inner, 0)
        return 0
    lax.fori_loop(0, EC, _row, 0)


def _zero_acc_slice(rows_v, acc_sh, row0):
    _fill_rows(rows_v, 0.0)

    def _zero(k, _):
        pltpu.sync_copy(rows_v, acc_sh.at[pl.ds(row0 + k * RC, RC)])
        return 0
    lax.fori_loop(0, NRC, _zero, 0)


def _copy_out_slice(rows_v, acc_sh, out_hbm, cid, row0):
    def _out(k, _):
        r = row0 + k * RC
        pltpu.sync_copy(acc_sh.at[pl.ds(r, RC)], rows_v)
        pltpu.sync_copy(rows_v, out_hbm.at[cid, pl.ds(r, RC)])
        return 0
    lax.fori_loop(0, NRC, _out, 0)


def _sc_body(z_hbm, src_hbm, dst_hbm, agg_out,
             srcA_v, dstA_v, srcB_v, dstB_v, rowsA_v, rowsB_v, acc_sh,
             semA, semB):
    cid = lax.axis_index("c")
    sid = lax.axis_index("s")
    wid = sid * NC + cid
    row0 = sid * RPT
    HG = FULL_K // 2

    _zero_acc_slice(rowsA_v, acc_sh, row0)
    plsc.subcore_barrier()

    def _load_and_issue(q, src_v, dst_v, rows_v, sem):
        base = q * EC
        pltpu.sync_copy(src_hbm.at[pl.ds(base, EC)], src_v.at[0])
        pltpu.sync_copy(dst_hbm.at[pl.ds(base, EC)], dst_v.at[0])
        pltpu.async_copy(z_hbm.at[src_v.at[0]], rows_v, sem)

    def _wait_and_scatter(src_v, dst_v, rows_v, sem):
        pltpu.make_async_copy(z_hbm.at[src_v.at[0]], rows_v, sem).wait()
        pltpu.sync_copy(rows_v, acc_sh.at[dst_v.at[0]], add=True)

    _load_and_issue(wid * FULL_K, srcA_v, dstA_v, rowsA_v, semA)

    @pl.loop(0, HG)
    def _pipe(g):
        c0 = wid * FULL_K + 2 * g
        _load_and_issue(c0 + 1, srcB_v, dstB_v, rowsB_v, semB)
        _wait_and_scatter(srcA_v, dstA_v, rowsA_v, semA)

        @pl.when(g < HG - 1)
        def _():
            _load_and_issue(c0 + 2, srcA_v, dstA_v, rowsA_v, semA)

        @pl.when((g == HG - 1) & (wid < TAIL))
        def _():
            _load_and_issue(NW * FULL_K + wid, srcA_v, dstA_v, rowsA_v, semA)

        _wait_and_scatter(srcB_v, dstB_v, rowsB_v, semB)

    @pl.when(wid < TAIL)
    def _():
        _wait_and_scatter(srcA_v, dstA_v, rowsA_v, semA)

    plsc.subcore_barrier()
    _copy_out_slice(rowsA_v, acc_sh, agg_out, cid, row0)


def _sc_count_body(dst_hbm, cnt_out, dst_v, rows_v, acc_sh):
    cid = lax.axis_index("c")
    sid = lax.axis_index("s")
    wid = sid * NC + cid
    row0 = sid * RPT

    _zero_acc_slice(rows_v, acc_sh, row0)
    _fill_rows(rows_v, 1.0)
    plsc.subcore_barrier()

    def _chunk_body(q):
        pltpu.sync_copy(dst_hbm.at[pl.ds(q * EC, EC)], dst_v.at[0])
        pltpu.sync_copy(rows_v, acc_sh.at[dst_v.at[0]], add=True)

    def _chunk(k, _):
        _chunk_body(wid * FULL_K + k)
        return 0
    lax.fori_loop(0, FULL_K, _chunk, 0)

    @pl.when(wid < TAIL)
    def _():
        _chunk_body(NW * FULL_K + wid)

    plsc.subcore_barrier()
    _copy_out_slice(rows_v, acc_sh, cnt_out, cid, row0)


def _sc_mesh():
    return plsc.VectorSubcoreMesh(
        core_axis_name="c", subcore_axis_name="s",
        num_cores=NC, num_subcores=NS,
    )


@functools.cache
def _sc_kernel():
    return pl.kernel(
        _sc_body,
        out_type=jax.ShapeDtypeStruct((NC, NP, H), jnp.float32),
        mesh=_sc_mesh(),
        scratch_types=[
            pltpu.VMEM((1, EC), jnp.int32),
            pltpu.VMEM((1, EC), jnp.int32),
            pltpu.VMEM((1, EC), jnp.int32),
            pltpu.VMEM((1, EC), jnp.int32),
            pltpu.VMEM((EC, H), jnp.float32),
            pltpu.VMEM((EC, H), jnp.float32),
            pltpu.VMEM_SHARED((NP, H), jnp.float32),
            pltpu.SemaphoreType.DMA,
            pltpu.SemaphoreType.DMA,
        ],
    )


@functools.cache
def _sc_count_kernel():
    return pl.kernel(
        _sc_count_body,
        out_type=jax.ShapeDtypeStruct((NC, NP, H), jnp.float32),
        mesh=_sc_mesh(),
        scratch_types=[
            pltpu.VMEM((1, EC), jnp.int32),
            pltpu.VMEM((EC, H), jnp.float32),
            pltpu.VMEM_SHARED((NP, H), jnp.float32),
        ],
    )


def _sc_aggregate(z, src, dst):
    return _sc_kernel()(z, src, dst)


def _sc_count(dst):
    return _sc_count_kernel()(dst)


def _enc_body(x_ref, w1_ref, b1_ref, w2_ref, b2_ref, o_ref):
    h = jnp.dot(x_ref[...], w1_ref[...], preferred_element_type=jnp.float32)
    h = jnp.maximum(h + b1_ref[...], 0.0)
    o_ref[...] = (
        jnp.dot(h, w2_ref[...], preferred_element_type=jnp.float32)
        + b2_ref[...]
    )


def _encode(xin, w1, b1, w2, b2):
    return pl.pallas_call(
        _enc_body,
        grid=(NB,),
        in_specs=[
            pl.BlockSpec((BR, xin.shape[1]), lambda i: (i, 0)),
            pl.BlockSpec(w1.shape, lambda i: (0, 0)),
            pl.BlockSpec(b1.shape, lambda i: (0, 0)),
            pl.BlockSpec(w2.shape, lambda i: (0, 0)),
            pl.BlockSpec(b2.shape, lambda i: (0, 0)),
        ],
        out_specs=pl.BlockSpec((BR, H), lambda i: (i, 0)),
        out_shape=jax.ShapeDtypeStruct((N, H), jnp.float32),
    )(xin, w1, b1, w2, b2)


def _mean_agg(p_ref, c_ref):
    cnt = c_ref[0][:, 0:1] + c_ref[1][:, 0:1]
    recip = 1.0 / jnp.maximum(cnt, 1.0)
    return (p_ref[0] + p_ref[1]) * recip


def _layer_body(p_ref, c_ref, z_ref, wl_ref, bl_ref, wr_ref, g_ref, b_ref,
                o_ref, t_buf, s_ref, q_ref):
    i = pl.program_id(0)

    @pl.when(i == 0)
    def _():
        s_ref[...] = jnp.zeros_like(s_ref)
        q_ref[...] = jnp.zeros_like(q_ref)

    @pl.when(i < NB)
    def _():
        agg = _mean_agg(p_ref, c_ref)
        t = (
            jnp.dot(agg, wl_ref[...], preferred_element_type=jnp.float32)
            + bl_ref[...]
            + jnp.dot(z_ref[...], wr_ref[...], preferred_element_type=jnp.float32)
        )
        t_buf[pl.ds(i * BR, BR), :] = t
        s_ref[0:1, :] += jnp.sum(t, axis=0, keepdims=True)
        q_ref[0:1, :] += jnp.sum(t * t, axis=0, keepdims=True)

    @pl.when(i >= NB)
    def _():
        j = i - NB
        t = t_buf[pl.ds(j * BR, BR), :]
        m = s_ref[0:1, :] * (1.0 / N)
        v = q_ref[0:1, :] * (1.0 / N) - m * m
        o_ref[...] = jnp.maximum(
            (t - m) * lax.rsqrt(v + 1e-5) * g_ref[...] + b_ref[...], 0.0
        )


def _layer(p, c, z, wl, bl, wr, g, b):
    return pl.pallas_call(
        _layer_body,
        grid=(2 * NB,),
        in_specs=[
            pl.BlockSpec((NC, BR, H), lambda i: (0, jnp.minimum(i, NB - 1), 0)),
            pl.BlockSpec((NC, BR, H), lambda i: (0, jnp.minimum(i, NB - 1), 0)),
            pl.BlockSpec((BR, H), lambda i: (jnp.minimum(i, NB - 1), 0)),
            pl.BlockSpec(wl.shape, lambda i: (0, 0)),
            pl.BlockSpec(bl.shape, lambda i: (0, 0)),
            pl.BlockSpec(wr.shape, lambda i: (0, 0)),
            pl.BlockSpec(g.shape, lambda i: (0, 0)),
            pl.BlockSpec(b.shape, lambda i: (0, 0)),
        ],
        out_specs=pl.BlockSpec((BR, H), lambda i: (jnp.maximum(i - NB, 0), 0)),
        out_shape=jax.ShapeDtypeStruct((N, H), jnp.float32),
        scratch_shapes=[
            pltpu.VMEM((N, H), jnp.float32),
            pltpu.VMEM((8, H), jnp.float32),
            pltpu.VMEM((8, H), jnp.float32),
        ],
    )(p, c, z, wl, bl, wr, g, b)


def _final_body(p_ref, c_ref, z_ref, wl_ref, bl_ref, wr_ref,
                w1_ref, b1_ref, w2_ref, b2_ref, o_ref):
    agg = _mean_agg(p_ref, c_ref)
    t = (
        jnp.dot(agg, wl_ref[...], preferred_element_type=jnp.float32)
        + bl_ref[...]
        + jnp.dot(z_ref[...], wr_ref[...], preferred_element_type=jnp.float32)
    )
    h = jnp.dot(t, w1_ref[...], preferred_element_type=jnp.float32)
    h = jnp.maximum(h + b1_ref[...], 0.0)
    o_ref[...] = (
        jnp.dot(h, w2_ref[...], preferred_element_type=jnp.float32)
        + b2_ref[...]
    )


def _final(p, c, z, wl, bl, wr, w1, b1, w2, b2):
    od = w2.shape[1]
    return pl.pallas_call(
        _final_body,
        grid=(NB,),
        in_specs=[
            pl.BlockSpec((NC, BR, H), lambda i: (0, i, 0)),
            pl.BlockSpec((NC, BR, H), lambda i: (0, i, 0)),
            pl.BlockSpec((BR, H), lambda i: (i, 0)),
            pl.BlockSpec(wl.shape, lambda i: (0, 0)),
            pl.BlockSpec(bl.shape, lambda i: (0, 0)),
            pl.BlockSpec(wr.shape, lambda i: (0, 0)),
            pl.BlockSpec(w1.shape, lambda i: (0, 0)),
            pl.BlockSpec(b1.shape, lambda i: (0, 0)),
            pl.BlockSpec(w2.shape, lambda i: (0, 0)),
            pl.BlockSpec(b2.shape, lambda i: (0, 0)),
        ],
        out_specs=pl.BlockSpec((BR, od), lambda i: (i, 0)),
        out_shape=jax.ShapeDtypeStruct((N, od), jnp.float32),
    )(p, c, z, wl, bl, wr, w1, b1, w2, b2)


def kernel(x, fx, enc_W1, enc_b1, enc_W2, enc_b2, sage_Wl, sage_bl, sage_Wr,
           bn_g, bn_b, dec_W1, dec_b1, dec_W2, dec_b2, geo):
    xin = jnp.concatenate([x[0], fx[0]], axis=-1)
    src = geo[0]
    dst = geo[1]
    z = _encode(xin, enc_W1, enc_b1.reshape(1, -1), enc_W2,
                enc_b2.reshape(1, -1))
    c = _sc_count(dst)
    n_layers = sage_Wl.shape[0] - 1
    for l in range(n_layers):
        p = _sc_aggregate(z, src, dst)
        z = _layer(p, c, z, sage_Wl[l], sage_bl[l].reshape(1, -1),
                   sage_Wr[l], bn_g[l].reshape(1, -1),
                   bn_b[l].reshape(1, -1))
    p = _sc_aggregate(z, src, dst)
    out = _final(p, c, z, sage_Wl[n_layers], sage_bl[n_layers].reshape(1, -1),
                 sage_Wr[n_layers], dec_W1, dec_b1.reshape(1, -1),
                 dec_W2, dec_b2.reshape(1, -1))
    return out[None]


# speedup vs baseline: 2.8171x; 1.0011x over previous
import functools

import jax
import jax.numpy as jnp
from jax import lax
from jax.experimental import pallas as pl
from jax.experimental.pallas import tpu as pltpu
from jax.experimental.pallas import tpu_sc as plsc

N = 10000
E = 320000
H = 128
NC, NS = 2, 16
NW = NC * NS
EC = 128
NCHUNK = E // EC
FULL_K = NCHUNK // NW
TAIL = NCHUNK - FULL_K * NW
NP = 10112
RPT = NP // NS
RC = 128
NRC = RPT // RC
REM = RPT - NRC * RC
CW = 16
NB = 10
BR = N // NB

def _fill_rows(rows_v, value):
    def _row(i, _):
        def _inner(j, _):
            rows_v[i, pl.ds(j * 16, 16)] = jnp.full((16,), value, jnp.float32)
            return 0
        lax.fori_loop(0, H // 16, _inner, 0)
        return 0
    lax.fori_loop(0, EC, _row, 0)


def _zero_acc_slice(rows_v, acc_sh, row0):
    _fill_rows(rows_v, 0.0)

    def _zero(k, _):
        pltpu.sync_copy(rows_v, acc_sh.at[pl.ds(row0 + k * RC, RC)])
        return 0
    lax.fori_loop(0, NRC, _zero, 0)
    pltpu.sync_copy(rows_v.at[pl.ds(0, REM)],
                    acc_sh.at[pl.ds(row0 + NRC * RC, REM)])


def _copy_out_slice(rows_v, acc_sh, out_hbm, cid, row0):
    def _out(k, _):
        r = row0 + k * RC
        pltpu.sync_copy(acc_sh.at[pl.ds(r, RC)], rows_v)
        pltpu.sync_copy(rows_v, out_hbm.at[cid, pl.ds(r, RC)])
        return 0
    lax.fori_loop(0, NRC, _out, 0)
    r = row0 + NRC * RC
    pltpu.sync_copy(acc_sh.at[pl.ds(r, REM)], rows_v.at[pl.ds(0, REM)])
    pltpu.sync_copy(rows_v.at[pl.ds(0, REM)], out_hbm.at[cid, pl.ds(r, REM)])


NPIPE = 3
NG = FULL_K // NPIPE


def _sc_body(z_hbm, src_hbm, dst_hbm, agg_out,
             src0_v, dst0_v, src1_v, dst1_v, src2_v, dst2_v,
             rows0_v, rows1_v, rows2_v, acc_sh, sem0, sem1, sem2):
    cid = lax.axis_index("c")
    sid = lax.axis_index("s")
    wid = sid * NC + cid
    row0 = sid * RPT
    srcs = [src0_v, src1_v, src2_v]
    dsts = [dst0_v, dst1_v, dst2_v]
    rows = [rows0_v, rows1_v, rows2_v]
    sems = [sem0, sem1, sem2]

    _zero_acc_slice(rows0_v, acc_sh, row0)
    plsc.subcore_barrier()

    def _load_and_issue(q, j):
        base = q * EC
        pltpu.sync_copy(src_hbm.at[pl.ds(base, EC)], srcs[j].at[0])
        pltpu.sync_copy(dst_hbm.at[pl.ds(base, EC)], dsts[j].at[0])
        pltpu.async_copy(z_hbm.at[srcs[j].at[0]], rows[j], sems[j])

    def _wait_and_scatter(j):
        pltpu.make_async_copy(z_hbm.at[srcs[j].at[0]], rows[j],
                              sems[j]).wait()
        pltpu.sync_copy(rows[j], acc_sh.at[dsts[j].at[0]], add=True)

    for j in range(NPIPE):
        _load_and_issue(wid * FULL_K + j, j)

    @pl.loop(0, NG)
    def _pipe(g):
        for j in range(NPIPE):
            c = NPIPE * g + j
            _wait_and_scatter(j)
            nxt = c + NPIPE

            @pl.when(nxt < FULL_K)
            def _():
                _load_and_issue(wid * FULL_K + nxt, j)

            @pl.when((nxt == FULL_K) & (wid < TAIL))
            def _():
                _load_and_issue(NW * FULL_K + wid, j)

    @pl.when(wid < TAIL)
    def _():
        _wait_and_scatter(0)

    plsc.subcore_barrier()
    _copy_out_slice(rows0_v, acc_sh, agg_out, cid, row0)


def _sc_count_body(dst_hbm, cnt_out, dst_v, rows_v, acc_sh):
    cid = lax.axis_index("c")
    sid = lax.axis_index("s")
    wid = sid * NC + cid
    row0 = sid * RPT

    _zero_acc_slice(rows_v, acc_sh, row0)
    _fill_rows(rows_v, 1.0)
    plsc.subcore_barrier()

    def _chunk_body(q):
        pltpu.sync_copy(dst_hbm.at[pl.ds(q * EC, EC)], dst_v.at[0])
        pltpu.sync_copy(rows_v, acc_sh.at[dst_v.at[0]], add=True)

    def _chunk(k, _):
        _chunk_body(wid * FULL_K + k)
        return 0
    lax.fori_loop(0, FULL_K, _chunk, 0)

    @pl.when(wid < TAIL)
    def _():
        _chunk_body(NW * FULL_K + wid)

    plsc.subcore_barrier()
    _copy_out_slice(rows_v, acc_sh, cnt_out, cid, row0)


def _sc_mesh():
    return plsc.VectorSubcoreMesh(
        core_axis_name="c", subcore_axis_name="s",
        num_cores=NC, num_subcores=NS,
    )


@functools.cache
def _sc_kernel():
    return pl.kernel(
        _sc_body,
        out_type=jax.ShapeDtypeStruct((NC, NP, H), jnp.float32),
        mesh=_sc_mesh(),
        scratch_types=[
            pltpu.VMEM((1, EC), jnp.int32),
            pltpu.VMEM((1, EC), jnp.int32),
            pltpu.VMEM((1, EC), jnp.int32),
            pltpu.VMEM((1, EC), jnp.int32),
            pltpu.VMEM((1, EC), jnp.int32),
            pltpu.VMEM((1, EC), jnp.int32),
            pltpu.VMEM((EC, H), jnp.float32),
            pltpu.VMEM((EC, H), jnp.float32),
            pltpu.VMEM((EC, H), jnp.float32),
            pltpu.VMEM_SHARED((NP, H), jnp.float32),
            pltpu.SemaphoreType.DMA,
            pltpu.SemaphoreType.DMA,
            pltpu.SemaphoreType.DMA,
        ],
    )


@functools.cache
def _sc_count_kernel():
    return pl.kernel(
        _sc_count_body,
        out_type=jax.ShapeDtypeStruct((NC, NP, H), jnp.float32),
        mesh=_sc_mesh(),
        scratch_types=[
            pltpu.VMEM((1, EC), jnp.int32),
            pltpu.VMEM((EC, H), jnp.float32),
            pltpu.VMEM_SHARED((NP, H), jnp.float32),
        ],
    )


def _sc_aggregate(z, src, dst):
    return _sc_kernel()(z, src, dst)


def _sc_count(dst):
    return _sc_count_kernel()(dst)


def _enc_body(x_ref, w1_ref, b1_ref, w2_ref, b2_ref, o_ref):
    h = jnp.dot(x_ref[...], w1_ref[...], preferred_element_type=jnp.float32)
    h = jnp.maximum(h + b1_ref[...], 0.0)
    o_ref[...] = (
        jnp.dot(h, w2_ref[...], preferred_element_type=jnp.float32)
        + b2_ref[...]
    )


def _encode(xin, w1, b1, w2, b2):
    return pl.pallas_call(
        _enc_body,
        grid=(NB,),
        in_specs=[
            pl.BlockSpec((BR, xin.shape[1]), lambda i: (i, 0)),
            pl.BlockSpec(w1.shape, lambda i: (0, 0)),
            pl.BlockSpec(b1.shape, lambda i: (0, 0)),
            pl.BlockSpec(w2.shape, lambda i: (0, 0)),
            pl.BlockSpec(b2.shape, lambda i: (0, 0)),
        ],
        out_specs=pl.BlockSpec((BR, H), lambda i: (i, 0)),
        out_shape=jax.ShapeDtypeStruct((N, H), jnp.float32),
    )(xin, w1, b1, w2, b2)


def _mean_agg(p_ref, c_ref):
    cnt = c_ref[0][:, 0:1] + c_ref[1][:, 0:1]
    recip = 1.0 / jnp.maximum(cnt, 1.0)
    return (p_ref[0] + p_ref[1]) * recip


def _layer_body(p_ref, c_ref, z_ref, wl_ref, bl_ref, wr_ref, g_ref, b_ref,
                o_ref, t_buf, s_ref, q_ref):
    i = pl.program_id(0)

    @pl.when(i == 0)
    def _():
        s_ref[...] = jnp.zeros_like(s_ref)
        q_ref[...] = jnp.zeros_like(q_ref)

    @pl.when(i < NB)
    def _():
        agg = _mean_agg(p_ref, c_ref)
        t = (
            jnp.dot(agg, wl_ref[...], preferred_element_type=jnp.float32)
            + bl_ref[...]
            + jnp.dot(z_ref[...], wr_ref[...], preferred_element_type=jnp.float32)
        )
        t_buf[pl.ds(i * BR, BR), :] = t
        s_ref[0:1, :] += jnp.sum(t, axis=0, keepdims=True)
        q_ref[0:1, :] += jnp.sum(t * t, axis=0, keepdims=True)

    @pl.when(i >= NB)
    def _():
        j = i - NB
        t = t_buf[pl.ds(j * BR, BR), :]
        m = s_ref[0:1, :] * (1.0 / N)
        v = q_ref[0:1, :] * (1.0 / N) - m * m
        o_ref[...] = jnp.maximum(
            (t - m) * lax.rsqrt(v + 1e-5) * g_ref[...] + b_ref[...], 0.0
        )


def _layer(p, c, z, wl, bl, wr, g, b):
    return pl.pallas_call(
        _layer_body,
        grid=(2 * NB,),
        in_specs=[
            pl.BlockSpec((NC, BR, H), lambda i: (0, jnp.minimum(i, NB - 1), 0)),
            pl.BlockSpec((NC, BR, H), lambda i: (0, jnp.minimum(i, NB - 1), 0)),
            pl.BlockSpec((BR, H), lambda i: (jnp.minimum(i, NB - 1), 0)),
            pl.BlockSpec(wl.shape, lambda i: (0, 0)),
            pl.BlockSpec(bl.shape, lambda i: (0, 0)),
            pl.BlockSpec(wr.shape, lambda i: (0, 0)),
            pl.BlockSpec(g.shape, lambda i: (0, 0)),
            pl.BlockSpec(b.shape, lambda i: (0, 0)),
        ],
        out_specs=pl.BlockSpec((BR, H), lambda i: (jnp.maximum(i - NB, 0), 0)),
        out_shape=jax.ShapeDtypeStruct((N, H), jnp.float32),
        scratch_shapes=[
            pltpu.VMEM((N, H), jnp.float32),
            pltpu.VMEM((8, H), jnp.float32),
            pltpu.VMEM((8, H), jnp.float32),
        ],
    )(p, c, z, wl, bl, wr, g, b)


def _final_body(p_ref, c_ref, z_ref, wl_ref, bl_ref, wr_ref,
                w1_ref, b1_ref, w2_ref, b2_ref, o_ref):
    agg = _mean_agg(p_ref, c_ref)
    t = (
        jnp.dot(agg, wl_ref[...], preferred_element_type=jnp.float32)
        + bl_ref[...]
        + jnp.dot(z_ref[...], wr_ref[...], preferred_element_type=jnp.float32)
    )
    h = jnp.dot(t, w1_ref[...], preferred_element_type=jnp.float32)
    h = jnp.maximum(h + b1_ref[...], 0.0)
    o_ref[...] = (
        jnp.dot(h, w2_ref[...], preferred_element_type=jnp.float32)
        + b2_ref[...]
    )


def _final(p, c, z, wl, bl, wr, w1, b1, w2, b2):
    od = w2.shape[1]
    return pl.pallas_call(
        _final_body,
        grid=(NB,),
        in_specs=[
            pl.BlockSpec((NC, BR, H), lambda i: (0, i, 0)),
            pl.BlockSpec((NC, BR, H), lambda i: (0, i, 0)),
            pl.BlockSpec((BR, H), lambda i: (i, 0)),
            pl.BlockSpec(wl.shape, lambda i: (0, 0)),
            pl.BlockSpec(bl.shape, lambda i: (0, 0)),
            pl.BlockSpec(wr.shape, lambda i: (0, 0)),
            pl.BlockSpec(w1.shape, lambda i: (0, 0)),
            pl.BlockSpec(b1.shape, lambda i: (0, 0)),
            pl.BlockSpec(w2.shape, lambda i: (0, 0)),
            pl.BlockSpec(b2.shape, lambda i: (0, 0)),
        ],
        out_specs=pl.BlockSpec((BR, od), lambda i: (i, 0)),
        out_shape=jax.ShapeDtypeStruct((N, od), jnp.float32),
    )(p, c, z, wl, bl, wr, w1, b1, w2, b2)


def kernel(x, fx, enc_W1, enc_b1, enc_W2, enc_b2, sage_Wl, sage_bl, sage_Wr,
           bn_g, bn_b, dec_W1, dec_b1, dec_W2, dec_b2, geo):
    xin = jnp.concatenate([x[0], fx[0]], axis=-1)
    src = geo[0]
    dst = geo[1]
    z = _encode(xin, enc_W1, enc_b1.reshape(1, -1), enc_W2,
                enc_b2.reshape(1, -1))
    c = _sc_count(dst)
    n_layers = sage_Wl.shape[0] - 1
    for l in range(n_layers):
        p = _sc_aggregate(z, src, dst)
        z = _layer(p, c, z, sage_Wl[l], sage_bl[l].reshape(1, -1),
                   sage_Wr[l], bn_g[l].reshape(1, -1),
                   bn_b[l].reshape(1, -1))
    p = _sc_aggregate(z, src, dst)
    out = _final(p, c, z, sage_Wl[n_layers], sage_bl[n_layers].reshape(1, -1),
                 sage_Wr[n_layers], dec_W1, dec_b1.reshape(1, -1),
                 dec_W2, dec_b2.reshape(1, -1))
    return out[None]
